# Initial kernel scaffold; baseline (speedup 1.0000x reference)
import jax
import jax.numpy as jnp
from jax.experimental import pallas as pl


def kernel(x, x_out, Q_all, V_all, slicer, negative_slicer):
    raise NotImplementedError("write your pallas kernel here")


# speedup vs baseline: 4.5276x; 4.5276x over previous
import functools

import jax
import jax.numpy as jnp
from jax import lax
from jax.experimental import pallas as pl
from jax.experimental.pallas import tpu as pltpu
from jax.experimental.pallas import tpu_sc as plsc

TOPK = 100
QVT_L2_WEIGHT = 0.001


def _sc_gather(Q_all, V_all, slicer, negative_slicer):
    info = plsc.get_sparse_core_info()
    nc, ns = info.num_cores, info.num_subcores
    nw = nc * ns
    n = slicer.shape[0]
    d = Q_all.shape[1]
    rpw = n // nw
    mesh = plsc.VectorSubcoreMesh(core_axis_name="c", subcore_axis_name="s")

    @functools.partial(
        pl.kernel,
        mesh=mesh,
        out_type=[jax.ShapeDtypeStruct((n, d), jnp.float32)] * 3,
        scratch_types=[
            pltpu.VMEM((rpw,), jnp.int32),
            pltpu.VMEM((rpw,), jnp.int32),
            pltpu.VMEM((rpw, d), jnp.float32),
            pltpu.VMEM((rpw, d), jnp.float32),
            pltpu.VMEM((rpw, d), jnp.float32),
            pltpu.SemaphoreType.DMA,
            pltpu.SemaphoreType.DMA,
            pltpu.SemaphoreType.DMA,
        ],
    )
    def gather_kernel(q_hbm, v_hbm, sl_hbm, neg_hbm, qs_out, qn_out, vn_out,
                      idx_a, idx_b, r0, r1, r2, sem0, sem1, sem2):
        wid = lax.axis_index("s") * nc + lax.axis_index("c")
        base = wid * rpw
        pltpu.sync_copy(sl_hbm.at[pl.ds(base, rpw)], idx_a)
        pltpu.sync_copy(neg_hbm.at[pl.ds(base, rpw)], idx_b)
        c0 = pltpu.async_copy(q_hbm.at[idx_a], r0, sem0)
        c1 = pltpu.async_copy(q_hbm.at[idx_b], r1, sem1)
        c2 = pltpu.async_copy(v_hbm.at[idx_b], r2, sem2)
        c0.wait()
        pltpu.sync_copy(r0, qs_out.at[pl.ds(base, rpw)])
        c1.wait()
        pltpu.sync_copy(r1, qn_out.at[pl.ds(base, rpw)])
        c2.wait()
        pltpu.sync_copy(r2, vn_out.at[pl.ds(base, rpw)])

    return gather_kernel(Q_all, V_all, slicer, negative_slicer)


def _reg_kernel(qn_ref, vn_ref, reg_ref, gq, gv, dsq, *, n_rows, n_tiles):
    t = pl.program_id(0)

    def nrm(m):
        s = jnp.sum(m * m, axis=1, keepdims=True)
        return m * (1.0 / jnp.maximum(jnp.sqrt(s), 1e-12))

    qnn = nrm(qn_ref[...])
    vnn = nrm(vn_ref[...])
    d = jnp.sum(qnn * vnn, axis=1, keepdims=True)

    @pl.when(t == 0)
    def _():
        gq[...] = jnp.zeros_like(gq)
        gv[...] = jnp.zeros_like(gv)
        dsq[...] = jnp.zeros_like(dsq)

    dims = (((0,), (0,)), ((), ()))
    gq[...] += lax.dot_general(qnn, qnn, dims, preferred_element_type=jnp.float32)
    gv[...] += lax.dot_general(vnn, vnn, dims, preferred_element_type=jnp.float32)
    dsq[...] += jnp.sum(d * d)

    @pl.when(t == n_tiles - 1)
    def _():
        n = jnp.float32(n_rows)
        numer = jnp.sum(gq[...] * gv[...]) / n - dsq[...] / n
        reg_ref[...] = QVT_L2_WEIGHT * jnp.maximum(numer, 0.0) / (n - 1.0)


def _tc_reg(qn, vn):
    n, d = qn.shape
    tile = 1024
    n_tiles = n // tile
    return pl.pallas_call(
        functools.partial(_reg_kernel, n_rows=n, n_tiles=n_tiles),
        grid=(n_tiles,),
        in_specs=[
            pl.BlockSpec((tile, d), lambda t: (t, 0)),
            pl.BlockSpec((tile, d), lambda t: (t, 0)),
        ],
        out_specs=pl.BlockSpec((1, 1), lambda t: (0, 0)),
        out_shape=jax.ShapeDtypeStruct((1, 1), jnp.float32),
        scratch_shapes=[
            pltpu.VMEM((d, d), jnp.float32),
            pltpu.VMEM((d, d), jnp.float32),
            pltpu.VMEM((1, 1), jnp.float32),
        ],
    )(qn, vn)


def _mm_kernel(x_ref, q_ref, nrm_ref, out_ref):
    qsn = q_ref[...] / nrm_ref[...]
    out_ref[...] = jnp.dot(x_ref[...], qsn,
                           preferred_element_type=jnp.float32)


def _tc_xq(x, qs_rows, nrmq):
    b, n = x.shape
    d = qs_rows.shape[1]
    bt = 256
    return pl.pallas_call(
        _mm_kernel,
        grid=(b // bt,),
        in_specs=[
            pl.BlockSpec((bt, n), lambda t: (t, 0)),
            pl.BlockSpec((n, d), lambda t: (0, 0)),
            pl.BlockSpec((n, 1), lambda t: (0, 0)),
        ],
        out_specs=pl.BlockSpec((bt, d), lambda t: (t, 0)),
        out_shape=jax.ShapeDtypeStruct((b, d), jnp.float32),
    )(x, qs_rows, nrmq)


def _score_topk_kernel(xq_ref, vn_ref, nv_ref, xout_ref, diag_ref, vals_ref,
                       inds_ref, s_ref, va, ia, *, bt, n, k_out):
    dims = (((1,), (1,)), ((), ()))
    vnn = vn_ref[...] / nv_ref[...]
    scores = lax.dot_general(xq_ref[...], vnn, dims,
                             preferred_element_type=jnp.float32)
    s_ref[...] = jnp.maximum(scores - xout_ref[...] * diag_ref[...], 0.0)

    iota = lax.broadcasted_iota(jnp.int32, (bt, n), 1)
    lane = lax.broadcasted_iota(jnp.int32, (bt, 128), 1)
    big = jnp.int32(2**30)

    def body(k, _):
        s = s_ref[...]
        m = jnp.max(s, axis=1, keepdims=True)
        cand = jnp.where(s == m, iota, big)
        idx = jnp.min(cand, axis=1, keepdims=True)
        s_ref[...] = jnp.where(cand == idx, -jnp.inf, s)
        va[...] = jnp.where(lane == k, m, va[...])
        ia[...] = jnp.where(lane == k, idx, ia[...])
        return 0

    lax.fori_loop(0, k_out, body, 0)
    vals_ref[...] = va[:, :k_out]
    inds_ref[...] = ia[:, :k_out]


def _tc_score_topk(xq, vn_rows, nv, x_out, diag_row):
    b, d = xq.shape
    n = vn_rows.shape[0]
    bt = 128
    grid = (b // bt,)
    return pl.pallas_call(
        functools.partial(_score_topk_kernel, bt=bt, n=n, k_out=TOPK),
        grid=grid,
        in_specs=[
            pl.BlockSpec((bt, d), lambda t: (t, 0)),
            pl.BlockSpec((n, d), lambda t: (0, 0)),
            pl.BlockSpec((n, 1), lambda t: (0, 0)),
            pl.BlockSpec((bt, n), lambda t: (t, 0)),
            pl.BlockSpec((1, n), lambda t: (0, 0)),
        ],
        out_specs=[
            pl.BlockSpec((bt, TOPK), lambda t: (t, 0)),
            pl.BlockSpec((bt, TOPK), lambda t: (t, 0)),
        ],
        out_shape=[
            jax.ShapeDtypeStruct((b, TOPK), jnp.float32),
            jax.ShapeDtypeStruct((b, TOPK), jnp.int32),
        ],
        scratch_shapes=[
            pltpu.VMEM((bt, n), jnp.float32),
            pltpu.VMEM((bt, 128), jnp.float32),
            pltpu.VMEM((bt, 128), jnp.int32),
        ],
    )(xq, vn_rows, nv, x_out, diag_row)


def kernel(x, x_out, Q_all, V_all, slicer, negative_slicer):
    qs_rows, qn_rows, vn_rows = _sc_gather(Q_all, V_all, slicer,
                                           negative_slicer)
    nq_s = jnp.maximum(jnp.sqrt(jnp.sum(qs_rows * qs_rows, -1, keepdims=True)), 1e-12)
    nq_n = jnp.maximum(jnp.sqrt(jnp.sum(qn_rows * qn_rows, -1, keepdims=True)), 1e-12)
    nv_n = jnp.maximum(jnp.sqrt(jnp.sum(vn_rows * vn_rows, -1, keepdims=True)), 1e-12)
    diag = jnp.sum((qn_rows / nq_n) * (vn_rows / nv_n), -1)
    reg = _tc_reg(qn_rows, vn_rows)
    xq = _tc_xq(x, qs_rows, nq_s)
    vals, inds = _tc_score_topk(xq, vn_rows, nv_n, x_out, diag.reshape(1, -1))
    return vals, inds, reg.reshape(())
